# Initial kernel scaffold; baseline (speedup 1.0000x reference)
import jax
import jax.numpy as jnp
from jax.experimental import pallas as pl


def kernel(inputs, tables):
    raise NotImplementedError("write your pallas kernel here")


# speedup vs baseline: 1.0307x; 1.0307x over previous
import functools

import jax
import jax.numpy as jnp
from jax import lax
from jax.experimental import pallas as pl
from jax.experimental.pallas import tpu as pltpu
from jax.experimental.pallas import tpu_sc as plsc

_N_FIELDS = 26
_VOCAB = 100000
_EDIM = 8
_BATCH = 16384

_INFO = plsc.get_sparse_core_info()
_NC = _INFO.num_cores
_NS = _INFO.num_subcores
_NW = _NC * _NS
_L = _INFO.num_lanes

_ROWS = _BATCH * _N_FIELDS
_IDX_MINOR = 128
_CHUNK = _N_FIELDS * _IDX_MINOR
_SLOTS = _ROWS // _CHUNK
_SLOTS_PER_W = _SLOTS // _NW


def _body(table_hbm, idx_hbm, off_hbm, out_hbm, off_v, idx_v, rows_v, sem):
    wid = lax.axis_index("s") * _NC + lax.axis_index("c")
    pltpu.sync_copy(off_hbm, off_v)

    for q in range(_SLOTS_PER_W):
        slot = wid * _SLOTS_PER_W + q
        pltpu.sync_copy(idx_hbm.at[slot], idx_v)

        def add_offsets(j, _):
            for k in range(_IDX_MINOR // _L):
                sl = pl.ds(k * _L, _L)
                idx_v[j, sl] = idx_v[j, sl] + off_v[j, sl]
            return _

        lax.fori_loop(0, _N_FIELDS, add_offsets, None)

        copies = []
        for j in range(_N_FIELDS):
            copies.append(
                pltpu.async_copy(
                    table_hbm.at[idx_v.at[j]],
                    rows_v.at[pl.ds(j * _IDX_MINOR, _IDX_MINOR)],
                    sem,
                )
            )
        for c in copies:
            c.wait()

        pltpu.sync_copy(rows_v, out_hbm.at[pl.ds(slot * _CHUNK, _CHUNK)])


@functools.partial(
    pl.kernel,
    mesh=plsc.VectorSubcoreMesh(core_axis_name="c", subcore_axis_name="s"),
    out_type=jax.ShapeDtypeStruct((_ROWS, _EDIM), jnp.float32),
    scratch_types=[
        pltpu.VMEM((_N_FIELDS, _IDX_MINOR), jnp.int32),
        pltpu.VMEM((_N_FIELDS, _IDX_MINOR), jnp.int32),
        pltpu.VMEM((_CHUNK, _EDIM), jnp.float32),
        pltpu.SemaphoreType.DMA,
    ],
    compiler_params=pltpu.CompilerParams(use_tc_tiling_on_sc=False),
)
def _sc_gather(table_hbm, idx_hbm, off_hbm, out_hbm, off_v, idx_v, rows_v, sem):
    _body(table_hbm, idx_hbm, off_hbm, out_hbm, off_v, idx_v, rows_v, sem)


def kernel(inputs, tables):
    table_flat = tables.reshape(_N_FIELDS * _VOCAB, _EDIM)
    idx3 = inputs.reshape(_SLOTS, _N_FIELDS, _IDX_MINOR)
    pos = jnp.arange(_CHUNK, dtype=jnp.int32).reshape(_N_FIELDS, _IDX_MINOR)
    off = (pos % _N_FIELDS) * _VOCAB
    out = _sc_gather(table_flat, idx3, off)
    return out.reshape(_BATCH, _N_FIELDS * _EDIM)


# speedup vs baseline: 4.7128x; 4.5724x over previous
import functools

import jax
import jax.numpy as jnp
from jax import lax
from jax.experimental import pallas as pl
from jax.experimental.pallas import tpu as pltpu
from jax.experimental.pallas import tpu_sc as plsc

_N_FIELDS = 26
_VOCAB = 100000
_EDIM = 8
_BATCH = 16384

_INFO = plsc.get_sparse_core_info()
_NC = _INFO.num_cores
_NS = _INFO.num_subcores
_NW = _NC * _NS
_L = _INFO.num_lanes

_TASKS = _N_FIELDS * _EDIM
_Q = -(-_TASKS // _NW)
_HALF = _BATCH // 2


def _body(table_hbm, idx_hbm, out_hbm, slice_v, idx_v, out_v):
    wid = lax.axis_index("s") * _NC + lax.axis_index("c")

    for q in range(_Q):
        t = q * _NW + wid

        @pl.when(t < _TASKS)
        def _run():
            f = t // _EDIM
            d = t % _EDIM
            pltpu.sync_copy(table_hbm.at[f, d], slice_v)
            for h in range(2):
                pltpu.sync_copy(idx_hbm.at[f, pl.ds(h * _HALF, _HALF)], idx_v)

                def gather(i, _):
                    sl = pl.ds(i * _L, _L)
                    out_v[sl] = plsc.load_gather(slice_v, [idx_v[sl]])
                    return _

                lax.fori_loop(0, _HALF // _L, gather, None)
                pltpu.sync_copy(out_v, out_hbm.at[t, pl.ds(h * _HALF, _HALF)])


@functools.partial(
    pl.kernel,
    mesh=plsc.VectorSubcoreMesh(core_axis_name="c", subcore_axis_name="s"),
    out_type=jax.ShapeDtypeStruct((_N_FIELDS * _EDIM, _BATCH), jnp.float32),
    scratch_types=[
        pltpu.VMEM((_VOCAB,), jnp.float32),
        pltpu.VMEM((_HALF,), jnp.int32),
        pltpu.VMEM((_HALF,), jnp.float32),
    ],
    compiler_params=pltpu.CompilerParams(
        use_tc_tiling_on_sc=False, needs_layout_passes=False
    ),
)
def _sc_gather(table_hbm, idx_hbm, out_hbm, slice_v, idx_v, out_v):
    _body(table_hbm, idx_hbm, out_hbm, slice_v, idx_v, out_v)


def kernel(inputs, tables):
    table_t = jnp.transpose(tables, (0, 2, 1))
    idx_t = inputs.T
    out_t = _sc_gather(table_t, idx_t)
    return out_t.T


# speedup vs baseline: 9.9257x; 2.1061x over previous
import functools

import jax
import jax.numpy as jnp
from jax import lax
from jax.experimental import pallas as pl
from jax.experimental.pallas import tpu as pltpu
from jax.experimental.pallas import tpu_sc as plsc

_N_FIELDS = 26
_VOCAB = 100000
_EDIM = 8
_BATCH = 16384

_INFO = plsc.get_sparse_core_info()
_NC = _INFO.num_cores
_NS = _INFO.num_subcores
_NW = _NC * _NS
_L = _INFO.num_lanes

_TASKS = _N_FIELDS * _EDIM
_Q = -(-_TASKS // _NW)
_HALF = _BATCH // 2


def _body(table_hbm, idx_hbm, out_hbm, slice_v, idx_v, out_v):
    wid = lax.axis_index("s") * _NC + lax.axis_index("c")

    for q in range(_Q):
        t = q * _NW + wid

        @pl.when(t < _TASKS)
        def _run():
            f = t // _EDIM
            d = t % _EDIM
            pltpu.sync_copy(table_hbm.at[f, d], slice_v)
            for h in range(2):
                pltpu.sync_copy(idx_hbm.at[f, pl.ds(h * _HALF, _HALF)], idx_v)

                def gather(i, _):
                    sl = pl.ds(i * _L, _L)
                    out_v[sl] = plsc.load_gather(slice_v, [idx_v[sl]])
                    return _

                lax.fori_loop(0, _HALF // _L, gather, None)
                pltpu.sync_copy(out_v, out_hbm.at[t, pl.ds(h * _HALF, _HALF)])


@functools.partial(
    pl.kernel,
    mesh=plsc.VectorSubcoreMesh(core_axis_name="c", subcore_axis_name="s"),
    out_type=jax.ShapeDtypeStruct((_N_FIELDS * _EDIM, _BATCH), jnp.float32),
    scratch_types=[
        pltpu.VMEM((_VOCAB,), jnp.float32),
        pltpu.VMEM((_HALF,), jnp.int32),
        pltpu.VMEM((_HALF,), jnp.float32),
    ],
    compiler_params=pltpu.CompilerParams(
        use_tc_tiling_on_sc=True, needs_layout_passes=False
    ),
)
def _sc_gather(table_hbm, idx_hbm, out_hbm, slice_v, idx_v, out_v):
    _body(table_hbm, idx_hbm, out_hbm, slice_v, idx_v, out_v)


def kernel(inputs, tables):
    table_t = jnp.transpose(tables, (0, 2, 1))
    idx_t = inputs.T
    out_t = _sc_gather(table_t, idx_t)
    return out_t.T


# speedup vs baseline: 14.0104x; 1.4115x over previous
import functools

import jax
import jax.numpy as jnp
from jax import lax
from jax.experimental import pallas as pl
from jax.experimental.pallas import tpu as pltpu
from jax.experimental.pallas import tpu_sc as plsc

_N_FIELDS = 26
_VOCAB = 100000
_EDIM = 8
_BATCH = 16384

_INFO = plsc.get_sparse_core_info()
_NC = _INFO.num_cores
_NS = _INFO.num_subcores
_NW = _NC * _NS
_L = _INFO.num_lanes

_TASKS = _N_FIELDS * _EDIM
_Q = -(-_TASKS // _NW)
_HALF = _BATCH // 2


def _run_task(table_hbm, idx_hbm, out_hbm, slice_v, idx_v, out_v, t, halves):
    f = t // _EDIM
    d = t % _EDIM
    pltpu.sync_copy(table_hbm.at[f, d], slice_v)
    for h in halves:
        pltpu.sync_copy(idx_hbm.at[f, pl.ds(h * _HALF, _HALF)], idx_v)

        @plsc.parallel_loop(0, _HALF // _L, 1, unroll=8)
        def gather(i):
            sl = pl.ds(i * _L, _L)
            out_v[sl] = plsc.load_gather(slice_v, [idx_v[sl]])

        pltpu.sync_copy(out_v, out_hbm.at[t, pl.ds(h * _HALF, _HALF)])


def _body(table_hbm, idx_hbm, out_hbm, slice_v, idx_v, out_v):
    wid = lax.axis_index("s") * _NC + lax.axis_index("c")

    for q in range(6):
        _run_task(table_hbm, idx_hbm, out_hbm, slice_v, idx_v, out_v,
                  q * _NW + wid, (0, 1))
    _run_task(table_hbm, idx_hbm, out_hbm, slice_v, idx_v, out_v,
              6 * _NW + lax.rem(wid, 16), (lax.div(wid, 16),))


@functools.partial(
    pl.kernel,
    mesh=plsc.VectorSubcoreMesh(core_axis_name="c", subcore_axis_name="s"),
    out_type=jax.ShapeDtypeStruct((_N_FIELDS * _EDIM, _BATCH), jnp.float32),
    scratch_types=[
        pltpu.VMEM((_VOCAB,), jnp.float32),
        pltpu.VMEM((_HALF,), jnp.int32),
        pltpu.VMEM((_HALF,), jnp.float32),
    ],
    compiler_params=pltpu.CompilerParams(
        use_tc_tiling_on_sc=True, needs_layout_passes=False
    ),
)
def _sc_gather(table_hbm, idx_hbm, out_hbm, slice_v, idx_v, out_v):
    _body(table_hbm, idx_hbm, out_hbm, slice_v, idx_v, out_v)


def kernel(inputs, tables):
    table_t = jnp.transpose(tables, (0, 2, 1))
    idx_t = inputs.T
    out_t = _sc_gather(table_t, idx_t)
    return out_t.T
